# Initial kernel scaffold; baseline (speedup 1.0000x reference)
import jax
import jax.numpy as jnp
from jax.experimental import pallas as pl


def kernel(x, edge_index, batch, params):
    raise NotImplementedError("write your pallas kernel here")


# speedup vs baseline: 2.6178x; 2.6178x over previous
import functools

import jax
import jax.numpy as jnp
from jax import lax
from jax.experimental import pallas as pl
from jax.experimental.pallas import tpu as pltpu
from jax.experimental.pallas import tpu_sc as plsc

N_NODES = 10000
N_GRAPHS = 256
N_EDGES = 160000
EPS = 1e-5

NC, NS, LANES = 2, 16, 16
NW = NC * NS
BN_ = 160
NCHUNK = 64
NPAD = NCHUNK * BN_
EB = N_EDGES // NW
K = 16
LE = N_EDGES + 64

_MESH = plsc.VectorSubcoreMesh(
    core_axis_name="c", subcore_axis_name="s", num_cores=NC, num_subcores=NS)
_SC_PARAMS = pltpu.CompilerParams(needs_layout_passes=False)


def _i32(v):
    return jnp.int32(v)


_GDN = lax.GatherDimensionNumbers(
    offset_dims=(), collapsed_slice_dims=(0,), start_index_map=(0,))


def _dyn_gather(x, idx):
    return lax.gather(x, idx[:, None], _GDN, (1,),
                      mode=lax.GatherScatterMode.PROMISE_IN_BOUNDS)


def _wid():
    return lax.axis_index("s") * _i32(NC) + lax.axis_index("c")


def _make_agg(W):
    @functools.partial(
        pl.kernel,
        out_type=jax.ShapeDtypeStruct((NPAD, W), jnp.float32),
        mesh=_MESH,
        scratch_types=[
            pltpu.VMEM((80,), jnp.int32),
            pltpu.VMEM((K,), jnp.int32),
            pltpu.VMEM((K,), jnp.int32),
            pltpu.VMEM((K, W), jnp.float32),
            pltpu.VMEM((BN_, W), jnp.float32),
            pltpu.SemaphoreType.DMA,
        ],
        compiler_params=_SC_PARAMS,
    )
    def agg(hs_hbm, row_hbm, col_hbm, bnd_hbm, out_hbm,
            bnd_v, idx_v, col_v, rows_v, acc_v, sem):
        wid = _wid()
        pltpu.sync_copy(bnd_hbm, bnd_v)

        def chunk(ci, carry):
            c = wid + ci * _i32(NW)
            n0 = c * _i32(BN_)

            def zero(i, cz):
                for f in range(W // LANES):
                    acc_v[i, pl.ds(f * LANES, LANES)] = jnp.zeros(
                        (LANES,), jnp.float32)
                return cz

            lax.fori_loop(_i32(0), _i32(BN_), zero, _i32(0))

            bv = bnd_v[pl.ds(c, LANES)]
            e0 = bv[0]
            e1 = bv[1]
            e0a = (e0 // _i32(8)) * _i32(8)
            nit = (e1 - e0a + _i32(K - 1)) // _i32(K)

            fidx = [jnp.arange(LANES, dtype=jnp.int32) + _i32(f * LANES)
                    for f in range(W // LANES)]

            def ebatch(i, cb):
                e = e0a + i * _i32(K)
                pltpu.sync_copy(row_hbm.at[pl.ds(e, K)], idx_v)
                pltpu.sync_copy(col_hbm.at[pl.ds(e, K)], col_v)
                pltpu.async_copy(hs_hbm.at[idx_v], rows_v, sem).wait()
                cv = col_v[pl.ds(0, K)] - n0
                ev = jnp.arange(LANES, dtype=jnp.int32) + e

                def edge(j, ce):
                    jv = jnp.full((LANES,), j, jnp.int32)
                    rv = _dyn_gather(cv, jv)
                    ejv = _dyn_gather(ev, jv)
                    mv = jnp.logical_and(ejv >= e0, ejv < e1)
                    for f in range(W // LANES):
                        plsc.addupdate_scatter(
                            acc_v, [rv, fidx[f]],
                            rows_v[j, pl.ds(f * LANES, LANES)], mask=mv)
                    return ce

                lax.fori_loop(_i32(0), _i32(K), edge, _i32(0))
                return cb

            lax.fori_loop(_i32(0), nit, ebatch, _i32(0))
            pltpu.sync_copy(acc_v, out_hbm.at[pl.ds(n0, BN_)])
            return carry

        lax.fori_loop(_i32(0), _i32(NCHUNK // NW), chunk, _i32(0))

    return agg


_AGG = {128: _make_agg(128), 256: _make_agg(256), 512: _make_agg(512)}


def _pad_rows(a):
    return jnp.pad(a, ((0, NPAD - a.shape[0]), (0, 0)))


def _relu_bn(z, g, b):
    y = jax.nn.relu(z)
    yr = y[:N_NODES]
    m = yr.mean(axis=0)
    v = yr.var(axis=0)
    return g * (y - m) / jnp.sqrt(v + EPS) + b


def kernel(x, edge_index, batch, params):
    p = params
    row = edge_index[0].astype(jnp.int32)
    col = edge_index[1].astype(jnp.int32)
    order = jnp.argsort(col)
    row_s = row[order]
    col_s = col[order]
    row_p = jnp.pad(row_s, (0, LE - N_EDGES))
    col_p = jnp.pad(col_s, (0, LE - N_EDGES))
    bnd = jnp.searchsorted(
        col_s, (jnp.arange(65) * BN_).astype(jnp.int32)).astype(jnp.int32)
    bnd = jnp.pad(bnd, (0, 80 - 65))

    order_r = jnp.argsort(row)
    row_rs = row[order_r]
    row_rs_p = jnp.pad(row_rs, (0, LE - N_EDGES))
    rbnd = jnp.searchsorted(
        row_rs, (jnp.arange(65) * BN_).astype(jnp.int32)).astype(jnp.int32)
    rbnd = jnp.pad(rbnd, (0, 80 - 65))
    ones_h = jnp.ones((NPAD, 128), jnp.float32)
    deg = 1.0 + _AGG[128](ones_h, row_rs_p, row_rs_p, rbnd)[:, 0]
    dinv = jnp.where(jnp.arange(NPAD) < N_NODES, deg ** -0.5, 0.0)[:, None]

    def edge_sum(hs, W):
        return _AGG[W](hs, row_p, col_p, bnd)

    x128 = _pad_rows(jnp.pad(x, ((0, 0), (0, 99))))
    W1 = jnp.pad(p['W1'], ((0, 99), (0, 0)))

    xs = dinv * x128
    t1 = dinv * (edge_sum(xs, 128) + xs)
    g1 = _relu_bn(t1 @ W1 + p['b1'], p['bn1_g'], p['bn1_b'])

    zs2 = dinv * (g1 @ p['W2'])
    g2 = _relu_bn(dinv * (edge_sum(zs2, 512) + zs2) + p['b2'],
                  p['bn2_g'], p['bn2_b'])

    zs3 = dinv * (g2 @ p['W3'])
    g3 = _relu_bn(dinv * (edge_sum(zs3, 256) + zs3) + p['b3'],
                  p['bn3_g'], p['bn3_b'])

    gs3 = dinv * g3
    t4 = dinv * (edge_sum(gs3, 256) + gs3)
    g4 = _relu_bn(t4 @ p['W4'] + p['b4'], p['bn4_g'], p['bn4_b'])

    gs4 = dinv * g4
    t5 = dinv * (edge_sum(gs4, 512) + gs4)
    g5 = _relu_bn(t5 @ p['W5'] + p['b5'], p['bn5_g'], p['bn5_b'])[:N_NODES]

    gate = g5 @ p['gate_W'] + p['gate_b']
    gmax = jax.ops.segment_max(gate, batch, num_segments=N_GRAPHS)
    gmax = jnp.where(jnp.isfinite(gmax), gmax, 0.0)
    e = jnp.exp(gate - gmax[batch])
    denom = jax.ops.segment_sum(e, batch, num_segments=N_GRAPHS)
    d = denom[batch]
    alpha = e / jnp.where(d > 0, d, 1.0)
    h = jax.ops.segment_sum(alpha * g5, batch, num_segments=N_GRAPHS)
    h = jax.nn.relu(h @ p['fc2_W'] + p['fc2_b'])
    h = jax.nn.relu(h @ p['fc3_W'] + p['fc3_b'])
    return h @ p['fc4_W'] + p['fc4_b']
